# baseline (speedup 1.0000x reference)
import functools

import jax
import jax.numpy as jnp
from jax import lax
from jax.experimental import pallas as pl
from jax.experimental.pallas import tpu as pltpu
from jax.experimental.pallas import tpu_sc as plsc

B = 4096
H = 50
N = B * H
D = 64
NC = 2
NS = 16
NW = NC * NS
CH = 128
PER_W = N // NW
NCHUNK = PER_W // CH
L = 16
G = CH // L
V_CLS, V_SUB, V_WT = 30, 300, 5


@functools.partial(
    pl.kernel,
    out_type=tuple(jax.ShapeDtypeStruct((N, D), jnp.float32) for _ in range(4)),
    mesh=plsc.VectorSubcoreMesh(core_axis_name="c", subcore_axis_name="s"),
    compiler_params=pltpu.CompilerParams(use_tc_tiling_on_sc=False,
                                         needs_layout_passes=False),
    scratch_types=[
        pltpu.VMEM((NCHUNK, CH), jnp.int32),
        pltpu.VMEM((2, 4, CH), jnp.int32),
        pltpu.VMEM((2, 4, CH), jnp.int32),
        pltpu.VMEM((2, 4, CH, D), jnp.float32),
        pltpu.VMEM((V_CLS, D), jnp.float32),
        pltpu.VMEM((V_SUB, D), jnp.float32),
        pltpu.VMEM((V_WT, D), jnp.float32),
        pltpu.SemaphoreType.DMA((2,)),
        pltpu.SemaphoreType.DMA((2,)),
        pltpu.SemaphoreType.DMA,
    ],
)
def _sc_lookup(jobs, mflat, tloc, tcls, tsub, twt,
               o0, o1, o2, o3, idx_v, mix_v, fid_v, rows_v,
               cls_v, sub_v, wt_v, sem_m, sem_l, sem_s):
    wid = lax.axis_index("s") * NC + lax.axis_index("c")
    base = wid * PER_W
    outs = (o0, o1, o2, o3)
    iota = lax.iota(jnp.int32, L)

    def compute_mix(k):
        buf = lax.rem(k, 2)
        for g in range(G):
            jobs16 = idx_v[k, pl.ds(g * L, L)]
            j4 = jobs16 * 4
            for f in range(4):
                mix_v[buf, f, pl.ds(g * L, L)] = j4 + f

    def meta_copies(k):
        buf = lax.rem(k, 2)
        return [pltpu.make_async_copy(mflat.at[mix_v.at[buf, f]],
                                      fid_v.at[buf, f], sem_m.at[buf])
                for f in range(4)]

    def loc_copy(k):
        buf = lax.rem(k, 2)
        return pltpu.make_async_copy(tloc.at[fid_v.at[buf, 0]],
                                     rows_v.at[buf, 0], sem_l.at[buf])

    def small_gathers(k):
        buf = lax.rem(k, 2)
        for f, tbl in ((1, cls_v), (2, sub_v), (3, wt_v)):
            dst = rows_v.at[buf, f]

            def g_body(g, carry):
                vfid = fid_v[buf, f, pl.ds(g * L, L)]
                rows16 = iota + g * L
                for c in range(D):
                    colc = jnp.full((L,), c, jnp.int32)
                    v = plsc.load_gather(tbl, [vfid, colc])
                    plsc.store_scatter(dst, [rows16, colc], v)
                return carry

            lax.fori_loop(0, G, g_body, 0)

    def store_copies(k):
        buf = lax.rem(k, 2)
        return [pltpu.make_async_copy(rows_v.at[buf, f],
                                      outs[f].at[pl.ds(base + k * CH, CH)],
                                      sem_s)
                for f in range(4)]

    def fire(copies):
        for c in copies:
            c.start()

    def drain(copies):
        for c in copies:
            c.wait()

    pltpu.sync_copy(jobs.at[wid], idx_v)
    pltpu.sync_copy(tcls, cls_v)
    pltpu.sync_copy(tsub, sub_v)
    pltpu.sync_copy(twt, wt_v)

    compute_mix(0)
    fire(meta_copies(0))
    compute_mix(1)
    fire(meta_copies(1))
    drain(meta_copies(0))
    loc_copy(0).start()
    small_gathers(0)

    def chunk_body(k, carry):

        @pl.when(k + 2 < NCHUNK)
        def _():
            with jax.named_scope("p_mix"):
                compute_mix(k + 2)
            fire(meta_copies(k + 2))

        @pl.when(k >= 1)
        def _():
            with jax.named_scope("p_storewait"):
                drain(store_copies(k - 1))

        @pl.when(k + 1 < NCHUNK)
        def _():
            with jax.named_scope("p_metawait"):
                drain(meta_copies(k + 1))
            loc_copy(k + 1).start()
            with jax.named_scope("p_small"):
                small_gathers(k + 1)

        with jax.named_scope("p_locwait"):
            loc_copy(k).wait()
        fire(store_copies(k))
        return carry

    lax.fori_loop(0, NCHUNK, chunk_body, 0)
    drain(store_copies(NCHUNK - 1))


def kernel(job_ids, metadata_table, loc_emb, cls_emb, sub_emb, wt_emb):
    jobs = job_ids.reshape(NW, NCHUNK, CH).astype(jnp.int32)
    mflat = metadata_table.reshape(4 * 1000000)
    outs = _sc_lookup(jobs, mflat, loc_emb, cls_emb, sub_emb, wt_emb)
    return tuple(o.reshape(B, H, D) for o in outs)


# speedup vs baseline: 2.4765x; 2.4765x over previous
import functools

import jax
import jax.numpy as jnp
from jax import lax
from jax.experimental import pallas as pl
from jax.experimental.pallas import tpu as pltpu
from jax.experimental.pallas import tpu_sc as plsc

B = 4096
H = 50
N = B * H
D = 64
NC = 2
NS = 16
NW = NC * NS
CH = 128
PER_W = N // NW
NCHUNK = PER_W // CH
L = 16
G = CH // L
V_CLS, V_SUB, V_WT = 30, 300, 5


@functools.partial(
    pl.kernel,
    out_type=tuple(jax.ShapeDtypeStruct((N, D), jnp.float32) for _ in range(4)),
    mesh=plsc.VectorSubcoreMesh(core_axis_name="c", subcore_axis_name="s"),
    compiler_params=pltpu.CompilerParams(use_tc_tiling_on_sc=False,
                                         needs_layout_passes=False),
    scratch_types=[
        pltpu.VMEM((NCHUNK, CH), jnp.int32),
        pltpu.VMEM((2, 4, CH), jnp.int32),
        pltpu.VMEM((2, 4, CH, D), jnp.float32),
        pltpu.VMEM((V_CLS, D), jnp.float32),
        pltpu.VMEM((V_SUB, D), jnp.float32),
        pltpu.VMEM((V_WT, D), jnp.float32),
        pltpu.SemaphoreType.DMA((2,)),
        pltpu.SemaphoreType.DMA((2,)),
        pltpu.SemaphoreType.DMA,
    ],
)
def _sc_lookup(jobs, mt, tloc, tcls, tsub, twt,
               o0, o1, o2, o3, idx_v, fid_v, rows_v,
               cls_v, sub_v, wt_v, sem_m, sem_l, sem_s):
    wid = lax.axis_index("s") * NC + lax.axis_index("c")
    base = wid * PER_W
    outs = (o0, o1, o2, o3)
    iota = lax.iota(jnp.int32, L)

    def meta_copies(k):
        buf = lax.rem(k, 2)
        return [pltpu.make_async_copy(mt.at[f].at[idx_v.at[k]],
                                      fid_v.at[buf, f], sem_m.at[buf])
                for f in range(4)]

    def loc_copy(k):
        buf = lax.rem(k, 2)
        return pltpu.make_async_copy(tloc.at[fid_v.at[buf, 0]],
                                     rows_v.at[buf, 0], sem_l.at[buf])

    def small_gathers(k):
        buf = lax.rem(k, 2)
        for f, tbl in ((1, cls_v), (2, sub_v), (3, wt_v)):
            dst = rows_v.at[buf, f]

            @plsc.parallel_loop(0, G, unroll=1)
            def g_body(g):
                vfid = fid_v[buf, f, pl.ds(g * L, L)]
                rows16 = iota + g * L
                for c in range(D):
                    colc = jnp.full((L,), c, jnp.int32)
                    v = plsc.load_gather(tbl, [vfid, colc])
                    plsc.store_scatter(dst, [rows16, colc], v)

    def store_copies(k):
        buf = lax.rem(k, 2)
        return [pltpu.make_async_copy(rows_v.at[buf, f],
                                      outs[f].at[pl.ds(base + k * CH, CH)],
                                      sem_s)
                for f in range(4)]

    def fire(copies):
        for c in copies:
            c.start()

    def drain(copies):
        for c in copies:
            c.wait()

    pltpu.sync_copy(jobs.at[wid], idx_v)
    pltpu.sync_copy(tcls, cls_v)
    pltpu.sync_copy(tsub, sub_v)
    pltpu.sync_copy(twt, wt_v)

    fire(meta_copies(0))
    fire(meta_copies(1))
    drain(meta_copies(0))
    loc_copy(0).start()
    small_gathers(0)

    def chunk_body(k, carry):

        @pl.when(k + 2 < NCHUNK)
        def _():
            fire(meta_copies(k + 2))

        @pl.when(k >= 1)
        def _():
            with jax.named_scope("p_storewait"):
                drain(store_copies(k - 1))

        @pl.when(k + 1 < NCHUNK)
        def _():
            with jax.named_scope("p_metawait"):
                drain(meta_copies(k + 1))
            loc_copy(k + 1).start()
            with jax.named_scope("p_small"):
                small_gathers(k + 1)

        with jax.named_scope("p_locwait"):
            loc_copy(k).wait()
        fire(store_copies(k))
        return carry

    lax.fori_loop(0, NCHUNK, chunk_body, 0)
    drain(store_copies(NCHUNK - 1))


def kernel(job_ids, metadata_table, loc_emb, cls_emb, sub_emb, wt_emb):
    jobs = job_ids.reshape(NW, NCHUNK, CH).astype(jnp.int32)
    mt = metadata_table.T
    outs = _sc_lookup(jobs, mt, loc_emb, cls_emb, sub_emb, wt_emb)
    return tuple(o.reshape(B, H, D) for o in outs)
